# Initial kernel scaffold; baseline (speedup 1.0000x reference)
import functools

import jax
import jax.numpy as jnp
from jax import lax
from jax.experimental import pallas as pl
from jax.experimental.pallas import tpu as pltpu
from jax.experimental.pallas import tpu_sc as plsc

B, T = 16384, 200
D = 64
L = B * T
NC, NS = 2, 16
NW = NC * NS
PER_W = L // NW
G = 128
CB = 4
C = G * CB
NCH = PER_W // C


def _make_kernel():
    mesh = plsc.VectorSubcoreMesh(core_axis_name="c", subcore_axis_name="s")

    @functools.partial(
        pl.kernel,
        mesh=mesh,
        out_type=jax.ShapeDtypeStruct((L, D), jnp.float32),
        scratch_types=[
            pltpu.VMEM((CB, G), jnp.int32),
            pltpu.VMEM((C, D), jnp.float32),
            pltpu.SemaphoreType.DMA,
        ],
    )
    def k(idx_hbm, table_hbm, out_hbm, idx_v, rows_v, sem):
        wid = lax.axis_index("s") * NC + lax.axis_index("c")
        base = wid * PER_W

        def body(i, carry):
            off = base + i * C
            pltpu.sync_copy(idx_hbm.at[pl.ds(off, C)], idx_v)
            copies = []
            for j in range(CB):
                copies.append(pltpu.async_copy(
                    table_hbm.at[idx_v.at[j]],
                    rows_v.at[pl.ds(j * G, G)],
                    sem,
                ))
            for cp in copies:
                cp.wait()
            pltpu.sync_copy(rows_v, out_hbm.at[pl.ds(off, C)])
            return carry

        lax.fori_loop(0, NCH, body, 0)

    return k


_gather = _make_kernel()


def kernel(indices, table):
    idx_flat = indices.astype(jnp.int32).reshape(L)
    out = _gather(idx_flat, table)
    return out.reshape(B, T, D)


# speedup vs baseline: 4.7324x; 4.7324x over previous
import functools

import jax
import jax.numpy as jnp
from jax import lax
from jax.experimental import pallas as pl
from jax.experimental.pallas import tpu as pltpu
from jax.experimental.pallas import tpu_sc as plsc

B, T = 16384, 200
D = 64
L = B * T
NC, NS = 2, 16
NW = NC * NS
PER_W = L // NW
G = 128
CB = 4
C = G * CB
NCH = PER_W // C


def _make_kernel():
    mesh = plsc.VectorSubcoreMesh(core_axis_name="c", subcore_axis_name="s")

    @functools.partial(
        pl.kernel,
        mesh=mesh,
        out_type=jax.ShapeDtypeStruct((L, D), jnp.float32),
        compiler_params=pltpu.CompilerParams(use_tc_tiling_on_sc=False),
        scratch_types=[
            pltpu.VMEM((CB, G), jnp.int32),
            pltpu.VMEM((C, D), jnp.float32),
            pltpu.SemaphoreType.DMA,
        ],
    )
    def k(idx_hbm, table_hbm, out_hbm, idx_v, rows_v, sem):
        wid = lax.axis_index("s") * NC + lax.axis_index("c")
        base = wid * PER_W

        row0 = wid * (PER_W // G)

        def body(i, carry):
            off = base + i * C
            pltpu.sync_copy(idx_hbm.at[pl.ds(row0 + i * CB, CB)], idx_v)
            copies = []
            for j in range(CB):
                copies.append(pltpu.async_copy(
                    table_hbm.at[idx_v.at[j]],
                    rows_v.at[pl.ds(j * G, G)],
                    sem,
                ))
            for cp in copies:
                cp.wait()
            pltpu.sync_copy(rows_v, out_hbm.at[pl.ds(off, C)])
            return carry

        lax.fori_loop(0, NCH, body, 0)

    return k


_gather = _make_kernel()


def kernel(indices, table):
    idx_2d = indices.astype(jnp.int32).reshape(L // G, G)
    out = _gather(idx_2d, table)
    return out.reshape(B, T, D)


# speedup vs baseline: 5.1187x; 1.0816x over previous
import functools

import jax
import jax.numpy as jnp
from jax import lax
from jax.experimental import pallas as pl
from jax.experimental.pallas import tpu as pltpu
from jax.experimental.pallas import tpu_sc as plsc

B, T = 16384, 200
D = 64
L = B * T
NC, NS = 2, 16
NW = NC * NS
PER_W = L // NW
G = 128
CB = 4
C = G * CB
NCH = PER_W // C
NBUF = 2


def _make_kernel():
    mesh = plsc.VectorSubcoreMesh(core_axis_name="c", subcore_axis_name="s")

    @functools.partial(
        pl.kernel,
        mesh=mesh,
        out_type=jax.ShapeDtypeStruct((L, D), jnp.float32),
        compiler_params=pltpu.CompilerParams(use_tc_tiling_on_sc=False),
        scratch_types=[
            pltpu.VMEM((NBUF, CB, G), jnp.int32),
            pltpu.VMEM((NBUF, C, D), jnp.float32),
            pltpu.SemaphoreType.DMA((NBUF,)),
            pltpu.SemaphoreType.DMA((NBUF,)),
            pltpu.SemaphoreType.DMA((NBUF,)),
        ],
    )
    def k(idx_hbm, table_hbm, out_hbm, idx_v, rows_v, sem_i, sem_g, sem_o):
        wid = lax.axis_index("s") * NC + lax.axis_index("c")
        base = wid * PER_W
        row0 = wid * (PER_W // G)

        def idx_copy(g, b):
            row = row0 + lax.min(g, NCH - 1) * CB
            return pltpu.make_async_copy(
                idx_hbm.at[pl.ds(row, CB)], idx_v.at[b], sem_i.at[b])

        def gather(b, j):
            return pltpu.make_async_copy(
                table_hbm.at[idx_v.at[b, j]],
                rows_v.at[b, pl.ds(j * G, G)],
                sem_g.at[b])

        def out_copy(g, b):
            return pltpu.make_async_copy(
                rows_v.at[b], out_hbm.at[pl.ds(base + g * C, C)], sem_o.at[b])

        for b in range(NBUF):
            idx_copy(b, b).start()

        def body(t, carry):
            for b in range(NBUF):
                g = t * NBUF + b
                idx_copy(g, b).wait()

                @pl.when(t > 0)
                def _():
                    out_copy(g, b).wait()

                for j in range(CB):
                    gather(b, j).start()
                for j in range(CB):
                    gather(b, j).wait()
                idx_copy(g + NBUF, b).start()
                out_copy(g, b).start()
            return carry

        lax.fori_loop(0, NCH // NBUF, body, 0)

        for b in range(NBUF):
            idx_copy(NCH - 1, b).wait()
            out_copy(NCH - NBUF + b, b).wait()

    return k


_gather_kernel = _make_kernel()


def kernel(indices, table):
    idx_2d = indices.astype(jnp.int32).reshape(L // G, G)
    out = _gather_kernel(idx_2d, table)
    return out.reshape(B, T, D)


# speedup vs baseline: 5.8155x; 1.1361x over previous
import functools

import jax
import jax.numpy as jnp
from jax import lax
from jax.experimental import pallas as pl
from jax.experimental.pallas import tpu as pltpu
from jax.experimental.pallas import tpu_sc as plsc

B, T = 16384, 200
D = 64
L = B * T
NC, NS = 2, 16
NW = NC * NS
PER_W = L // NW
G = 128
CB = 4
C = G * CB
NCH = PER_W // C
NBUF = 2


def _make_kernel():
    mesh = plsc.VectorSubcoreMesh(core_axis_name="c", subcore_axis_name="s")

    @functools.partial(
        pl.kernel,
        mesh=mesh,
        out_type=jax.ShapeDtypeStruct((L, D), jnp.float32),
        compiler_params=pltpu.CompilerParams(use_tc_tiling_on_sc=False),
        scratch_types=[
            pltpu.VMEM((NBUF, CB, G), jnp.int32),
            pltpu.VMEM((NBUF, C, D), jnp.float32),
            pltpu.VMEM_SHARED((8192, D), jnp.float32),
            pltpu.SemaphoreType.DMA((NBUF,)),
            pltpu.SemaphoreType.DMA((NBUF,)),
            pltpu.SemaphoreType.DMA((NBUF,)),
        ],
    )
    def k(idx_hbm, table_hbm, out_hbm, idx_v, rows_v, table_sh,
          sem_i, sem_g, sem_o):
        wid = lax.axis_index("s") * NC + lax.axis_index("c")
        base = wid * PER_W
        row0 = wid * (PER_W // G)

        @pl.when(lax.axis_index("s") == 0)
        def _():
            pltpu.sync_copy(table_hbm, table_sh)

        plsc.subcore_barrier()

        def idx_copy(g, b):
            row = row0 + lax.min(g, NCH - 1) * CB
            return pltpu.make_async_copy(
                idx_hbm.at[pl.ds(row, CB)], idx_v.at[b], sem_i.at[b])

        def gather(b, j):
            return pltpu.make_async_copy(
                table_sh.at[idx_v.at[b, j]],
                rows_v.at[b, pl.ds(j * G, G)],
                sem_g.at[b])

        def out_copy(g, b):
            return pltpu.make_async_copy(
                rows_v.at[b], out_hbm.at[pl.ds(base + g * C, C)], sem_o.at[b])

        for b in range(NBUF):
            idx_copy(b, b).start()

        def body(t, carry):
            for b in range(NBUF):
                g = t * NBUF + b
                idx_copy(g, b).wait()

                @pl.when(t > 0)
                def _():
                    out_copy(g, b).wait()

                for j in range(CB):
                    gather(b, j).start()
                for j in range(CB):
                    gather(b, j).wait()
                idx_copy(g + NBUF, b).start()
                out_copy(g, b).start()
            return carry

        lax.fori_loop(0, NCH // NBUF, body, 0)

        for b in range(NBUF):
            idx_copy(NCH - 1, b).wait()
            out_copy(NCH - NBUF + b, b).wait()

    return k


_gather_kernel = _make_kernel()


def kernel(indices, table):
    idx_2d = indices.astype(jnp.int32).reshape(L // G, G)
    out = _gather_kernel(idx_2d, table)
    return out.reshape(B, T, D)


# speedup vs baseline: 6.8224x; 1.1731x over previous
import functools

import jax
import jax.numpy as jnp
from jax import lax
from jax.experimental import pallas as pl
from jax.experimental.pallas import tpu as pltpu
from jax.experimental.pallas import tpu_sc as plsc

B, T = 16384, 200
D = 64
DP = 128
L = B * T
NC, NS = 2, 16
NW = NC * NS
PER_W = L // NW
G = 128
CB = 2
C = G * CB
KPB = 8
BLK = KPB * G
NBLK = PER_W // BLK
CPB = BLK // C
NBUF = 2


def _make_kernel():
    mesh = plsc.VectorSubcoreMesh(core_axis_name="c", subcore_axis_name="s")

    @functools.partial(
        pl.kernel,
        mesh=mesh,
        out_type=jax.ShapeDtypeStruct((L, DP), jnp.float32),
        scratch_types=[
            pltpu.VMEM((NBUF, KPB, G), jnp.int32),
            pltpu.VMEM((NBUF, C, DP), jnp.float32),
            pltpu.SemaphoreType.DMA((NBUF,)),
            pltpu.SemaphoreType.DMA((NBUF,)),
            pltpu.SemaphoreType.DMA((NBUF,)),
        ],
    )
    def k(idx_hbm, table_hbm, out_hbm, idx_v, rows_v, sem_i, sem_g, sem_o):
        wid = lax.axis_index("s") * NC + lax.axis_index("c")
        base = wid * PER_W
        blk0 = wid * NBLK

        def idx_copy(blk, bb):
            return pltpu.make_async_copy(
                idx_hbm.at[blk0 + lax.min(blk, NBLK - 1)],
                idx_v.at[bb], sem_i.at[bb])

        def gather(bb, r, b, j):
            return pltpu.make_async_copy(
                table_hbm.at[idx_v.at[bb, r]],
                rows_v.at[b, pl.ds(j * G, G)],
                sem_g.at[b])

        def out_copy(off, b):
            return pltpu.make_async_copy(
                rows_v.at[b], out_hbm.at[pl.ds(off, C)], sem_o.at[b])

        idx_copy(0, 0).start()

        def block_step(blk, bb):
            idx_copy(blk, bb).wait()
            idx_copy(blk + 1, 1 - bb).start()
            for c in range(CPB):
                b = c % NBUF
                g = blk * CPB + c
                if c >= NBUF:
                    out_copy(base, b).wait()
                else:
                    @pl.when(blk >= 1)
                    def _():
                        out_copy(base, b).wait()

                for j in range(CB):
                    gather(bb, c * CB + j, b, j).start()
                for j in range(CB):
                    gather(bb, c * CB + j, b, j).wait()
                out_copy(base + g * C, b).start()

        def body(t, carry):
            for par in range(2):
                block_step(t * 2 + par, par)
            return carry

        lax.fori_loop(0, NBLK // 2, body, 0)

        for b in range(NBUF):
            out_copy(base, b).wait()
        idx_copy(NBLK - 1, 0).wait()

    return k


_gather_kernel = _make_kernel()


def kernel(indices, table):
    idx_3d = indices.astype(jnp.int32).reshape(L // BLK, KPB, G)
    table_p = jnp.pad(table, ((0, 0), (0, DP - D)))
    out = _gather_kernel(idx_3d, table_p)
    return out[:, :D].reshape(B, T, D)
